# Initial kernel scaffold; baseline (speedup 1.0000x reference)
import jax
import jax.numpy as jnp
from jax.experimental import pallas as pl


def kernel(x, W_q, keys, weight_down, weight_up):
    raise NotImplementedError("write your pallas kernel here")


# speedup vs baseline: 3.2342x; 3.2342x over previous
import functools

import jax
import jax.numpy as jnp
from jax import lax
from jax.experimental import pallas as pl
from jax.experimental.pallas import tpu as pltpu
from jax.experimental.pallas import tpu_sc as plsc

DIM = 1024
HEADS = 8
NUM_KEYS = 256
DIM_KEY = 128
TOPK = 8
KTOT = HEADS * TOPK
N_TOKENS = 2048
TB = 256
NEG = -3.0e38


def _top8(s, width):
    iota = lax.broadcasted_iota(jnp.int32, s.shape, 1)
    vs, ix = [], []
    for _ in range(8):
        m = jnp.max(s, axis=1, keepdims=True)
        pos = jnp.min(jnp.where(s >= m, iota, width), axis=1, keepdims=True)
        vs.append(m)
        ix.append(pos)
        s = jnp.where(iota == pos, NEG, s)
    return jnp.concatenate(vs, axis=1), jnp.concatenate(ix, axis=1)


def _routing_body(x_ref, wq_ref, kt_ref, eidx_ref, coeff_ref):
    xb = x_ref[...]
    for h in range(HEADS):
        svals, sidxs = [], []
        for p in range(2):
            col = (p * HEADS + h) * DIM_KEY
            q_ph = jnp.dot(xb, wq_ref[:, col:col + DIM_KEY],
                           preferred_element_type=jnp.float32)
            sim = jnp.dot(q_ph, kt_ref[p, h],
                          preferred_element_type=jnp.float32)
            v, i = _top8(sim, NUM_KEYS)
            svals.append(v)
            sidxs.append(i)
        sx, sy = svals
        ixx, ixy = sidxs
        alls = jnp.concatenate([sx[:, i:i + 1] + sy for i in range(8)], axis=1)
        alli = jnp.concatenate(
            [ixx[:, i:i + 1] * NUM_KEYS + ixy for i in range(8)], axis=1)
        iota = lax.broadcasted_iota(jnp.int32, alls.shape, 1)
        s = alls
        vs, es = [], []
        for _ in range(8):
            m = jnp.max(s, axis=1, keepdims=True)
            pos = jnp.min(jnp.where(s >= m, iota, 64), axis=1, keepdims=True)
            sel = iota == pos
            e = jnp.sum(jnp.where(sel, alli, 0), axis=1, keepdims=True)
            vs.append(m)
            es.append(e)
            s = jnp.where(sel, NEG, s)
        v8 = jnp.concatenate(vs, axis=1)
        e8 = jnp.concatenate(es, axis=1)
        m8 = jnp.max(v8, axis=1, keepdims=True)
        ex = jnp.exp(v8 - m8)
        cf = ex / jnp.sum(ex, axis=1, keepdims=True)
        eidx_ref[:, h * 8:(h + 1) * 8] = e8
        coeff_ref[:, h * 8:(h + 1) * 8] = cf


def _routing(x2, W_q, keysT, interpret=False):
    grid = (N_TOKENS // TB,)
    return pl.pallas_call(
        _routing_body,
        grid=grid,
        in_specs=[
            pl.BlockSpec((TB, DIM), lambda i: (i, 0)),
            pl.BlockSpec((DIM, 2 * HEADS * DIM_KEY), lambda i: (0, 0)),
            pl.BlockSpec((2, HEADS, DIM_KEY, NUM_KEYS), lambda i: (0, 0, 0, 0)),
        ],
        out_specs=[
            pl.BlockSpec((TB, KTOT), lambda i: (i, 0)),
            pl.BlockSpec((TB, KTOT), lambda i: (i, 0)),
        ],
        out_shape=[
            jax.ShapeDtypeStruct((N_TOKENS, KTOT), jnp.int32),
            jax.ShapeDtypeStruct((N_TOKENS, KTOT), jnp.float32),
        ],
        interpret=interpret,
    )(x2, W_q, keysT)


def _gelu16(v):
    z = jnp.abs(v) * jnp.float32(0.7071067811865476)
    t = jnp.float32(1.0) / (jnp.float32(1.0) + jnp.float32(0.3275911) * z)
    poly = t * (jnp.float32(0.254829592)
                + t * (jnp.float32(-0.284496736)
                       + t * (jnp.float32(1.421413741)
                              + t * (jnp.float32(-1.453152027)
                                     + t * jnp.float32(1.061405429)))))
    erf_abs = jnp.float32(1.0) - poly * jnp.exp(-z * z)
    erf = jnp.where(v >= jnp.float32(0.0), erf_abs, -erf_abs)
    return v * jnp.float32(0.5) * (jnp.float32(1.0) + erf)


_NSUB = 32
_TPW = N_TOKENS // _NSUB
_NCH = DIM // 16


def _sc_body(x_hbm, eidx_hbm, coeff_hbm, wd_hbm, wu_hbm, out_hbm,
             idx_v, xrow_v, rows_v, crow_v, cg_v, acc_v, sem):
    wid = lax.axis_index("s") * 2 + lax.axis_index("c")
    iota16 = lax.iota(jnp.int32, 16)

    def token_body(i, carry):
        t = wid * _TPW + i
        pltpu.sync_copy(eidx_hbm.at[t], idx_v)
        pltpu.sync_copy(coeff_hbm.at[t], crow_v)
        pltpu.sync_copy(x_hbm.at[t], xrow_v)
        pltpu.async_copy(wd_hbm.at[idx_v], rows_v, sem).wait()

        for c in range(4):
            def pair_body(j2, hv):
                j = c * 16 + j2

                def d_body(dd, acc):
                    for u in range(8):
                        sl = pl.ds(pl.multiple_of(dd * 128 + u * 16, 16), 16)
                        acc = acc + rows_v[j, sl] * xrow_v[sl]
                    return acc

                acc = lax.fori_loop(0, _NCH // 8, d_body,
                                    jnp.zeros((16,), jnp.float32))
                s = jnp.sum(acc)
                return jnp.where(iota16 == j2, s, hv)

            hv = lax.fori_loop(0, 16, pair_body, jnp.zeros((16,), jnp.float32))
            cg_v[pl.ds(c * 16, 16)] = crow_v[pl.ds(c * 16, 16)] * _gelu16(hv)

        pltpu.async_copy(wu_hbm.at[idx_v], rows_v, sem).wait()
        for dd in range(_NCH):
            acc_v[pl.ds(dd * 16, 16)] = jnp.zeros((16,), jnp.float32)

        def pair2(j, carry2):
            base = pl.multiple_of((j // 16) * 16, 16)
            cvec = cg_v[pl.ds(base, 16)]
            csca = jnp.sum(jnp.where(iota16 == (j % 16), cvec,
                                     jnp.float32(0.0)))

            def d_body2(dd, c2):
                for u in range(8):
                    sl = pl.ds(pl.multiple_of(dd * 128 + u * 16, 16), 16)
                    plsc.addupdate(acc_v.at[sl], rows_v[j, sl] * csca)
                return c2

            return lax.fori_loop(0, _NCH // 8, d_body2, carry2)

        lax.fori_loop(0, KTOT, pair2, 0)
        pltpu.sync_copy(acc_v, out_hbm.at[t])
        return carry

    lax.fori_loop(0, _TPW, token_body, 0)


def _sc_moe(x2, eidx, coeff, weight_down, weight_up):
    mesh = plsc.VectorSubcoreMesh(core_axis_name="c", subcore_axis_name="s")
    f = functools.partial(
        pl.kernel,
        mesh=mesh,
        compiler_params=pltpu.CompilerParams(needs_layout_passes=False),
        out_type=jax.ShapeDtypeStruct((N_TOKENS, DIM), jnp.float32),
        scratch_types=[
            pltpu.VMEM((KTOT,), jnp.int32),
            pltpu.VMEM((DIM,), jnp.float32),
            pltpu.VMEM((KTOT, DIM), jnp.float32),
            pltpu.VMEM((KTOT,), jnp.float32),
            pltpu.VMEM((KTOT,), jnp.float32),
            pltpu.VMEM((DIM,), jnp.float32),
            pltpu.SemaphoreType.DMA,
        ],
    )(_sc_body)
    return f(x2, eidx, coeff, weight_down, weight_up)


def kernel(x, W_q, keys, weight_down, weight_up):
    b, n, d = x.shape
    x2 = x.reshape(n, d)
    keysT = jnp.transpose(keys, (2, 0, 3, 1))
    eidx, coeff = _routing(x2, W_q, keysT)
    out = _sc_moe(x2, eidx, coeff, weight_down, weight_up)
    return out.reshape(b, n, d)


# speedup vs baseline: 6.7736x; 2.0944x over previous
import functools

import jax
import jax.numpy as jnp
from jax import lax
from jax.experimental import pallas as pl
from jax.experimental.pallas import tpu as pltpu
from jax.experimental.pallas import tpu_sc as plsc

DIM = 1024
HEADS = 8
NUM_KEYS = 256
DIM_KEY = 128
TOPK = 8
KTOT = HEADS * TOPK
N_TOKENS = 2048
TB = 256
NEG = -3.0e38


def _top8(s, width):
    iota = lax.broadcasted_iota(jnp.int32, s.shape, 1)
    vs, ix = [], []
    for _ in range(8):
        m = jnp.max(s, axis=1, keepdims=True)
        pos = jnp.min(jnp.where(s >= m, iota, width), axis=1, keepdims=True)
        vs.append(m)
        ix.append(pos)
        s = jnp.where(iota == pos, NEG, s)
    return jnp.concatenate(vs, axis=1), jnp.concatenate(ix, axis=1)


def _routing_body(x_ref, wq_ref, kt_ref, eidx_ref, coeff_ref):
    xb = x_ref[...]
    for h in range(HEADS):
        svals, sidxs = [], []
        for p in range(2):
            col = (p * HEADS + h) * DIM_KEY
            q_ph = jnp.dot(xb, wq_ref[:, col:col + DIM_KEY],
                           preferred_element_type=jnp.float32)
            sim = jnp.dot(q_ph, kt_ref[p, h],
                          preferred_element_type=jnp.float32)
            v, i = _top8(sim, NUM_KEYS)
            svals.append(v)
            sidxs.append(i)
        sx, sy = svals
        ixx, ixy = sidxs
        alls = jnp.concatenate([sx[:, i:i + 1] + sy for i in range(8)], axis=1)
        alli = jnp.concatenate(
            [ixx[:, i:i + 1] * NUM_KEYS + ixy for i in range(8)], axis=1)
        iota = lax.broadcasted_iota(jnp.int32, alls.shape, 1)
        s = alls
        vs, es = [], []
        for _ in range(8):
            m = jnp.max(s, axis=1, keepdims=True)
            pos = jnp.min(jnp.where(s >= m, iota, 64), axis=1, keepdims=True)
            sel = iota == pos
            e = jnp.sum(jnp.where(sel, alli, 0), axis=1, keepdims=True)
            vs.append(m)
            es.append(e)
            s = jnp.where(sel, NEG, s)
        v8 = jnp.concatenate(vs, axis=1)
        e8 = jnp.concatenate(es, axis=1)
        m8 = jnp.max(v8, axis=1, keepdims=True)
        ex = jnp.exp(v8 - m8)
        cf = ex / jnp.sum(ex, axis=1, keepdims=True)
        eidx_ref[:, h * 8:(h + 1) * 8] = e8
        coeff_ref[:, h * 8:(h + 1) * 8] = cf


def _routing(x2, W_q, keysT, interpret=False):
    grid = (N_TOKENS // TB,)
    return pl.pallas_call(
        _routing_body,
        grid=grid,
        in_specs=[
            pl.BlockSpec((TB, DIM), lambda i: (i, 0)),
            pl.BlockSpec((DIM, 2 * HEADS * DIM_KEY), lambda i: (0, 0)),
            pl.BlockSpec((2, HEADS, DIM_KEY, NUM_KEYS), lambda i: (0, 0, 0, 0)),
        ],
        out_specs=[
            pl.BlockSpec((TB, KTOT), lambda i: (i, 0)),
            pl.BlockSpec((TB, KTOT), lambda i: (i, 0)),
        ],
        out_shape=[
            jax.ShapeDtypeStruct((N_TOKENS, KTOT), jnp.int32),
            jax.ShapeDtypeStruct((N_TOKENS, KTOT), jnp.float32),
        ],
        interpret=interpret,
    )(x2, W_q, keysT)


def _gelu16(v):
    z = jnp.abs(v) * jnp.float32(0.7071067811865476)
    t = jnp.float32(1.0) / (jnp.float32(1.0) + jnp.float32(0.3275911) * z)
    poly = t * (jnp.float32(0.254829592)
                + t * (jnp.float32(-0.284496736)
                       + t * (jnp.float32(1.421413741)
                              + t * (jnp.float32(-1.453152027)
                                     + t * jnp.float32(1.061405429)))))
    erf_abs = jnp.float32(1.0) - poly * jnp.exp(-z * z)
    erf = jnp.where(v >= jnp.float32(0.0), erf_abs, -erf_abs)
    return v * jnp.float32(0.5) * (jnp.float32(1.0) + erf)


_NSUB = 32
_TPW = N_TOKENS // _NSUB
_NCH = DIM // 16
_HALF = KTOT // 2


def _sc_body(x_hbm, eidx_hbm, coeff_hbm, wd_hbm, wu_hbm, out_hbm,
             idx_all, cf_all, x8, out8, rowsA, rowsB, cg_v, semA, semB):
    wid = lax.axis_index("s") * 2 + lax.axis_index("c")
    iota16 = lax.iota(jnp.int32, 16)
    base_t = wid * _TPW

    def gather(tab, i, half, buf, sem):
        return pltpu.async_copy(tab.at[idx_all.at[i, pl.ds(half * _HALF,
                                                           _HALF)]], buf, sem)

    pltpu.sync_copy(eidx_hbm.at[pl.ds(base_t, _TPW)], idx_all)
    pltpu.sync_copy(coeff_hbm.at[pl.ds(base_t, _TPW)], cf_all)
    pltpu.sync_copy(x_hbm.at[pl.ds(base_t, 8)], x8)
    gather(wd_hbm, 0, 0, rowsA, semA)

    def dots_half(i, xloc, half, buf):
        for c in range(2):
            def g2_body(g2, hv):
                jb = c * 16 + g2 * 4

                def d_body(d, accs):
                    a0, a1, a2, a3 = accs
                    for u in range(8):
                        sl = pl.ds(pl.multiple_of(d * 128 + u * 16, 16), 16)
                        xv = x8[xloc, sl]
                        a0 = a0 + buf[jb + 0, sl] * xv
                        a1 = a1 + buf[jb + 1, sl] * xv
                        a2 = a2 + buf[jb + 2, sl] * xv
                        a3 = a3 + buf[jb + 3, sl] * xv
                    return (a0, a1, a2, a3)

                z = jnp.zeros((16,), jnp.float32)
                accs = plsc.parallel_loop(0, _NCH // 8, 1,
                                          carry=(z, z, z, z))(d_body)
                for q in range(4):
                    hv = jnp.where(iota16 == g2 * 4 + q, jnp.sum(accs[q]), hv)
                return hv

            hv = lax.fori_loop(0, 4, g2_body, jnp.zeros((16,), jnp.float32))
            ch = half * 2 + c
            cg_v[pl.ds(ch * 16, 16)] = (cf_all[i, pl.ds(ch * 16, 16)]
                                        * _gelu16(hv))

    def acc_half(i, xloc, half, buf):
        def j_body(j2, carry):
            j = half * _HALF + j2
            cbase = pl.multiple_of((j // 16) * 16, 16)
            cvec = cg_v[pl.ds(cbase, 16)]
            csca = jnp.sum(jnp.where(iota16 == lax.rem(j, 16), cvec,
                                     jnp.float32(0.0)))

            def d_body(d):
                for u in range(8):
                    sl = pl.ds(pl.multiple_of(d * 128 + u * 16, 16), 16)
                    plsc.addupdate(out8.at[xloc, sl], buf[j2, sl] * csca)

            plsc.parallel_loop(0, _NCH // 8, 1)(d_body)
            return carry

        lax.fori_loop(0, _HALF, j_body, 0)

    def token_body(i, carry):
        xloc = lax.rem(i, 8)

        @pl.when(jnp.logical_and(xloc == 0, i > 0))
        def _():
            pltpu.sync_copy(
                x_hbm.at[pl.ds(pl.multiple_of(base_t + i, 8), 8)], x8)

        gather(wd_hbm, i, 1, rowsB, semB)

        def z_body(d, c2):
            for u in range(8):
                sl = pl.ds(pl.multiple_of(d * 128 + u * 16, 16), 16)
                out8[xloc, sl] = jnp.zeros((16,), jnp.float32)
            return c2

        lax.fori_loop(0, _NCH // 8, z_body, 0)

        pltpu.make_async_copy(wd_hbm.at[idx_all.at[i, pl.ds(0, _HALF)]],
                              rowsA, semA).wait()
        dots_half(i, xloc, 0, rowsA)
        gather(wu_hbm, i, 0, rowsA, semA)

        pltpu.make_async_copy(wd_hbm.at[idx_all.at[i, pl.ds(_HALF, _HALF)]],
                              rowsB, semB).wait()
        dots_half(i, xloc, 1, rowsB)
        gather(wu_hbm, i, 1, rowsB, semB)

        pltpu.make_async_copy(wu_hbm.at[idx_all.at[i, pl.ds(0, _HALF)]],
                              rowsA, semA).wait()
        acc_half(i, xloc, 0, rowsA)

        @pl.when(i < _TPW - 1)
        def _():
            gather(wd_hbm, i + 1, 0, rowsA, semA)

        pltpu.make_async_copy(wu_hbm.at[idx_all.at[i, pl.ds(_HALF, _HALF)]],
                              rowsB, semB).wait()
        acc_half(i, xloc, 1, rowsB)

        @pl.when(xloc == 7)
        def _():
            pltpu.sync_copy(
                out8, out_hbm.at[pl.ds(pl.multiple_of(base_t + i - 7, 8), 8)])

        return carry

    lax.fori_loop(0, _TPW, token_body, 0)


def _sc_moe(x2, eidx, coeff, weight_down, weight_up):
    mesh = plsc.VectorSubcoreMesh(core_axis_name="c", subcore_axis_name="s")
    f = functools.partial(
        pl.kernel,
        mesh=mesh,
        compiler_params=pltpu.CompilerParams(needs_layout_passes=False),
        out_type=jax.ShapeDtypeStruct((N_TOKENS, DIM), jnp.float32),
        scratch_types=[
            pltpu.VMEM((_TPW, KTOT), jnp.int32),
            pltpu.VMEM((_TPW, KTOT), jnp.float32),
            pltpu.VMEM((8, DIM), jnp.float32),
            pltpu.VMEM((8, DIM), jnp.float32),
            pltpu.VMEM((_HALF, DIM), jnp.float32),
            pltpu.VMEM((_HALF, DIM), jnp.float32),
            pltpu.VMEM((KTOT,), jnp.float32),
            pltpu.SemaphoreType.DMA,
            pltpu.SemaphoreType.DMA,
        ],
    )(_sc_body)
    return f(x2, eidx, coeff, weight_down, weight_up)


def kernel(x, W_q, keys, weight_down, weight_up):
    b, n, d = x.shape
    x2 = x.reshape(n, d)
    keysT = jnp.transpose(keys, (2, 0, 3, 1))
    eidx, coeff = _routing(x2, W_q, keysT)
    out = _sc_moe(x2, eidx, coeff, weight_down, weight_up)
    return out.reshape(b, n, d)


# speedup vs baseline: 10.3635x; 1.5300x over previous
import functools

import jax
import jax.numpy as jnp
from jax import lax
from jax.experimental import pallas as pl
from jax.experimental.pallas import tpu as pltpu
from jax.experimental.pallas import tpu_sc as plsc

DIM = 1024
HEADS = 8
NUM_KEYS = 256
DIM_KEY = 128
TOPK = 8
KTOT = HEADS * TOPK
N_TOKENS = 2048
TB = 256
NEG = -3.0e38


def _top8(s, width):
    iota = lax.broadcasted_iota(jnp.int32, s.shape, 1).astype(jnp.float32)
    vs, ix = [], []
    for _ in range(8):
        m = jnp.max(s, axis=1, keepdims=True)
        pos = -jnp.max(jnp.where(s >= m, -iota, NEG), axis=1, keepdims=True)
        vs.append(m)
        ix.append(pos)
        s = jnp.where(iota == pos, NEG, s)
    return jnp.concatenate(vs, axis=1), jnp.concatenate(ix, axis=1)


def _routing_body(x_ref, wq_ref, kt_ref, eidx_ref, coeff_ref):
    xb = x_ref[...]
    for h in range(HEADS):
        svals, sidxs = [], []
        for p in range(2):
            col = (p * HEADS + h) * DIM_KEY
            q_ph = jnp.dot(xb, wq_ref[:, col:col + DIM_KEY],
                           preferred_element_type=jnp.float32)
            sim = jnp.dot(q_ph, kt_ref[p, h],
                          preferred_element_type=jnp.float32)
            v, i = _top8(sim, NUM_KEYS)
            svals.append(v)
            sidxs.append(i)
        sx, sy = svals
        ixx, ixy = sidxs
        alls = jnp.concatenate([sx[:, i:i + 1] + sy for i in range(8)], axis=1)
        alli = jnp.concatenate(
            [ixx[:, i:i + 1] * NUM_KEYS + ixy for i in range(8)], axis=1)
        iota = lax.broadcasted_iota(jnp.int32, alls.shape, 1).astype(jnp.float32)
        s = alls
        vs, es = [], []
        for _ in range(8):
            m = jnp.max(s, axis=1, keepdims=True)
            pos = -jnp.max(jnp.where(s >= m, -iota, NEG), axis=1,
                           keepdims=True)
            sel = iota == pos
            e = jnp.max(jnp.where(sel, alli, NEG), axis=1, keepdims=True)
            vs.append(m)
            es.append(e)
            s = jnp.where(sel, NEG, s)
        v8 = jnp.concatenate(vs, axis=1)
        e8 = jnp.concatenate(es, axis=1).astype(jnp.int32)
        m8 = jnp.max(v8, axis=1, keepdims=True)
        ex = jnp.exp(v8 - m8)
        cf = ex / jnp.sum(ex, axis=1, keepdims=True)
        eidx_ref[:, h * 8:(h + 1) * 8] = e8
        coeff_ref[:, h * 8:(h + 1) * 8] = cf


def _routing(x2, W_q, keysT, interpret=False):
    n_tok = x2.shape[0]
    grid = (n_tok // TB,)
    return pl.pallas_call(
        _routing_body,
        grid=grid,
        in_specs=[
            pl.BlockSpec((TB, DIM), lambda i: (i, 0)),
            pl.BlockSpec((DIM, 2 * HEADS * DIM_KEY), lambda i: (0, 0)),
            pl.BlockSpec((2, HEADS, DIM_KEY, NUM_KEYS), lambda i: (0, 0, 0, 0)),
        ],
        out_specs=[
            pl.BlockSpec((TB, KTOT), lambda i: (i, 0)),
            pl.BlockSpec((TB, KTOT), lambda i: (i, 0)),
        ],
        out_shape=[
            jax.ShapeDtypeStruct((n_tok, KTOT), jnp.int32),
            jax.ShapeDtypeStruct((n_tok, KTOT), jnp.float32),
        ],
        interpret=interpret,
    )(x2, W_q, keysT)


def _gelu16(v):
    z = jnp.abs(v) * jnp.float32(0.7071067811865476)
    t = jnp.float32(1.0) / (jnp.float32(1.0) + jnp.float32(0.3275911) * z)
    poly = t * (jnp.float32(0.254829592)
                + t * (jnp.float32(-0.284496736)
                       + t * (jnp.float32(1.421413741)
                              + t * (jnp.float32(-1.453152027)
                                     + t * jnp.float32(1.061405429)))))
    erf_abs = jnp.float32(1.0) - poly * jnp.exp(-z * z)
    erf = jnp.where(v >= jnp.float32(0.0), erf_abs, -erf_abs)
    return v * jnp.float32(0.5) * (jnp.float32(1.0) + erf)


_NSUB = 32
_NCH = DIM // 16
_HALF = KTOT // 2


def _make_sc_body(tpw):
  def _sc_body(x_hbm, eidx_hbm, coeff_hbm, wd_hbm, wu_hbm, out_hbm,
               idx_all, cf_all, x8, out8, rowsA, rowsB, cg_v, semA, semB):
    wid = lax.axis_index("s") * 2 + lax.axis_index("c")
    iota16 = lax.iota(jnp.int32, 16)
    base_t = wid * tpw

    def gather(tab, i, half, buf, sem):
        return pltpu.async_copy(tab.at[idx_all.at[i, pl.ds(half * _HALF,
                                                           _HALF)]], buf, sem)

    pltpu.sync_copy(eidx_hbm.at[pl.ds(base_t, tpw)], idx_all)
    pltpu.sync_copy(coeff_hbm.at[pl.ds(base_t, tpw)], cf_all)
    pltpu.sync_copy(x_hbm.at[pl.ds(base_t, 8)], x8)
    gather(wd_hbm, 0, 0, rowsA, semA)

    def dots_half(i, xloc, half, buf):
        for c in range(2):
            def g2_body(g2, hv):
                jb = c * 16 + g2 * 4

                def d_body(d, accs):
                    a0, a1, a2, a3 = accs
                    for u in range(8):
                        sl = pl.ds(pl.multiple_of(d * 128 + u * 16, 16), 16)
                        xv = x8[xloc, sl]
                        a0 = a0 + buf[jb + 0, sl] * xv
                        a1 = a1 + buf[jb + 1, sl] * xv
                        a2 = a2 + buf[jb + 2, sl] * xv
                        a3 = a3 + buf[jb + 3, sl] * xv
                    return (a0, a1, a2, a3)

                z = jnp.zeros((16,), jnp.float32)
                accs = plsc.parallel_loop(0, _NCH // 8, 1,
                                          carry=(z, z, z, z))(d_body)
                for q in range(4):
                    hv = jnp.where(iota16 == g2 * 4 + q, jnp.sum(accs[q]), hv)
                return hv

            hv = lax.fori_loop(0, 4, g2_body, jnp.zeros((16,), jnp.float32))
            ch = half * 2 + c
            cg_v[pl.ds(ch * 16, 16)] = (cf_all[i, pl.ds(ch * 16, 16)]
                                        * _gelu16(hv))

    def acc_half(i, xloc, half, buf):
        def j_body(j2, carry):
            j = half * _HALF + j2
            cbase = pl.multiple_of((j // 16) * 16, 16)
            cvec = cg_v[pl.ds(cbase, 16)]
            csca = jnp.sum(jnp.where(iota16 == lax.rem(j, 16), cvec,
                                     jnp.float32(0.0)))

            def d_body(d):
                for u in range(8):
                    sl = pl.ds(pl.multiple_of(d * 128 + u * 16, 16), 16)
                    plsc.addupdate(out8.at[xloc, sl], buf[j2, sl] * csca)

            plsc.parallel_loop(0, _NCH // 8, 1)(d_body)
            return carry

        lax.fori_loop(0, _HALF, j_body, 0)

    def token_body(i, carry):
        xloc = lax.rem(i, 8)

        @pl.when(jnp.logical_and(xloc == 0, i > 0))
        def _():
            pltpu.sync_copy(
                x_hbm.at[pl.ds(pl.multiple_of(base_t + i, 8), 8)], x8)

        gather(wd_hbm, i, 1, rowsB, semB)

        def z_body(d, c2):
            for u in range(8):
                sl = pl.ds(pl.multiple_of(d * 128 + u * 16, 16), 16)
                out8[xloc, sl] = jnp.zeros((16,), jnp.float32)
            return c2

        lax.fori_loop(0, _NCH // 8, z_body, 0)

        pltpu.make_async_copy(wd_hbm.at[idx_all.at[i, pl.ds(0, _HALF)]],
                              rowsA, semA).wait()
        dots_half(i, xloc, 0, rowsA)
        gather(wu_hbm, i, 0, rowsA, semA)

        pltpu.make_async_copy(wd_hbm.at[idx_all.at[i, pl.ds(_HALF, _HALF)]],
                              rowsB, semB).wait()
        dots_half(i, xloc, 1, rowsB)
        gather(wu_hbm, i, 1, rowsB, semB)

        pltpu.make_async_copy(wu_hbm.at[idx_all.at[i, pl.ds(0, _HALF)]],
                              rowsA, semA).wait()
        acc_half(i, xloc, 0, rowsA)

        @pl.when(i < tpw - 1)
        def _():
            gather(wd_hbm, i + 1, 0, rowsA, semA)

        pltpu.make_async_copy(wu_hbm.at[idx_all.at[i, pl.ds(_HALF, _HALF)]],
                              rowsB, semB).wait()
        acc_half(i, xloc, 1, rowsB)

        @pl.when(xloc == 7)
        def _():
            pltpu.sync_copy(
                out8, out_hbm.at[pl.ds(pl.multiple_of(base_t + i - 7, 8), 8)])

        return carry

    lax.fori_loop(0, tpw, token_body, 0)
  return _sc_body


def _sc_moe(x2, eidx, coeff, weight_down, weight_up):
    n_tok = x2.shape[0]
    tpw = n_tok // _NSUB
    mesh = plsc.VectorSubcoreMesh(core_axis_name="c", subcore_axis_name="s")
    f = functools.partial(
        pl.kernel,
        mesh=mesh,
        compiler_params=pltpu.CompilerParams(needs_layout_passes=False),
        out_type=jax.ShapeDtypeStruct((n_tok, DIM), jnp.float32),
        scratch_types=[
            pltpu.VMEM((tpw, KTOT), jnp.int32),
            pltpu.VMEM((tpw, KTOT), jnp.float32),
            pltpu.VMEM((8, DIM), jnp.float32),
            pltpu.VMEM((8, DIM), jnp.float32),
            pltpu.VMEM((_HALF, DIM), jnp.float32),
            pltpu.VMEM((_HALF, DIM), jnp.float32),
            pltpu.VMEM((KTOT,), jnp.float32),
            pltpu.SemaphoreType.DMA,
            pltpu.SemaphoreType.DMA,
        ],
    )(_make_sc_body(tpw))
    return f(x2, eidx, coeff, weight_down, weight_up)


N_CHUNKS = 4


def kernel(x, W_q, keys, weight_down, weight_up):
    b, n, d = x.shape
    x2 = x.reshape(n, d)
    keysT = jnp.transpose(keys, (2, 0, 3, 1))
    cs = n // N_CHUNKS
    outs = []
    for c in range(N_CHUNKS):
        xc = lax.slice_in_dim(x2, c * cs, (c + 1) * cs, axis=0)
        eidx, coeff = _routing(xc, W_q, keysT)
        outs.append(_sc_moe(xc, eidx, coeff, weight_down, weight_up))
    return jnp.concatenate(outs, axis=0).reshape(b, n, d)
